# Initial kernel scaffold; baseline (speedup 1.0000x reference)
import jax
import jax.numpy as jnp
from jax.experimental import pallas as pl


def kernel(x, edge_index, edge_attr, batch, W1, root1, b1, W2, root2, b2, lin1_w, lin1_b, lin2_w, lin2_b, lin3_w, lin3_b):
    raise NotImplementedError("write your pallas kernel here")


# speedup vs baseline: 1.9812x; 1.9812x over previous
import functools

import jax
import jax.numpy as jnp
from jax import lax
from jax.experimental import pallas as pl
from jax.experimental.pallas import tpu as pltpu
from jax.experimental.pallas import tpu_sc as plsc

N_NODES = 10000
N_EDGES = 320000
N_FEAT = 128
NHID = 128
N_CLASSES = 10
N_REL = 16
N_GRAPHS = 64

AUG = 144
CHUNK = 12288
SH_ROWS = 12416
TRASH = CHUNK
NPASS = 14
A_ROWS = NPASS * CHUNK
E_PAD = 327680
EDGE_BLK = 16000
NODE_BLK = 1000


def _seg_body(attr_ref, dst_ref, seg_ref):
    attr = attr_ref[...]
    m = jnp.max(attr, axis=-1, keepdims=True)
    idx = lax.broadcasted_iota(jnp.int32, attr.shape, 2)
    am = jnp.min(jnp.where(attr == m, idx, N_REL), axis=-1)
    seg_ref[...] = (am * N_NODES + dst_ref[0])[None]


def _edge_seg(edge_attr, dst):
    nblk = N_EDGES // EDGE_BLK
    return pl.pallas_call(
        _seg_body,
        grid=(nblk,),
        in_specs=[
            pl.BlockSpec((1, EDGE_BLK, N_REL), lambda i: (i, 0, 0)),
            pl.BlockSpec((1, 1, EDGE_BLK), lambda i: (i, 0, 0)),
        ],
        out_specs=pl.BlockSpec((1, 1, EDGE_BLK), lambda i: (i, 0, 0)),
        out_shape=jax.ShapeDtypeStruct((nblk, 1, EDGE_BLK), jnp.int32),
    )(edge_attr.reshape(nblk, EDGE_BLK, N_REL),
      dst.reshape(nblk, 1, EDGE_BLK)).reshape(N_EDGES)


def _sc_scatter_body(xa, src, seg, zs, a_out,
                     idx_v, seg_v, sidx_v, rows_v, shared, sem):
    c = lax.axis_index("c")
    s = lax.axis_index("s")
    zoff = s * (SH_ROWS // 16)
    ooff = s * (CHUNK // 16)
    for p in range(NPASS // 2):
        pid = p * 2 + c
        lo = pid * CHUNK
        pltpu.sync_copy(zs.at[pl.ds(zoff, SH_ROWS // 16)],
                        shared.at[pl.ds(zoff, SH_ROWS // 16)])
        plsc.subcore_barrier()

        def body(ch, carry):
            base = s * (E_PAD // 16) + ch * 128
            pltpu.sync_copy(seg.at[pl.ds(base, 128)], seg_v)
            pltpu.sync_copy(src.at[pl.ds(base, 128)], idx_v)
            for j in range(8):
                v = seg_v[pl.ds(j * 16, 16)]
                m = (v >= lo) & (v < lo + CHUNK)
                sidx_v[pl.ds(j * 16, 16)] = jnp.where(m, v - lo, TRASH)
            pltpu.async_copy(xa.at[idx_v], rows_v, sem).wait()
            pltpu.sync_copy(rows_v, shared.at[sidx_v], add=True)
            return carry

        lax.fori_loop(0, E_PAD // 16 // 128, body, 0)
        plsc.subcore_barrier()
        pltpu.sync_copy(shared.at[pl.ds(ooff, CHUNK // 16)],
                        a_out.at[pl.ds(lo + ooff, CHUNK // 16)])
        plsc.subcore_barrier()


@functools.partial(
    pl.kernel,
    out_type=jax.ShapeDtypeStruct((A_ROWS, AUG), jnp.float32),
    mesh=plsc.VectorSubcoreMesh(core_axis_name="c", subcore_axis_name="s"),
    compiler_params=pltpu.CompilerParams(use_tc_tiling_on_sc=False),
    scratch_types=[
        pltpu.VMEM((128,), jnp.int32),
        pltpu.VMEM((128,), jnp.int32),
        pltpu.VMEM((128,), jnp.int32),
        pltpu.VMEM((128, AUG), jnp.float32),
        pltpu.VMEM_SHARED((SH_ROWS, AUG), jnp.float32),
        pltpu.SemaphoreType.DMA,
    ],
)
def _sc_scatter(xa, src, seg, zs, a_out,
                idx_v, seg_v, sidx_v, rows_v, shared, sem):
    _sc_scatter_body(xa, src, seg, zs, a_out,
                     idx_v, seg_v, sidx_v, rows_v, shared, sem)


def _combine(x, A_ref, W_ref, root, b):
    acc = jnp.dot(x, root, preferred_element_type=jnp.float32) + b
    for r in range(N_REL):
        Ar = A_ref[r]
        cnt = jnp.maximum(Ar[:, NHID:NHID + 1], 1.0)
        acc = acc + jnp.dot(Ar[:, :NHID] / cnt, W_ref[r],
                            preferred_element_type=jnp.float32)
    return jnp.maximum(acc, 0.0)


def _augment(h):
    n = h.shape[0]
    return jnp.concatenate(
        [h, jnp.ones((n, 1), jnp.float32), jnp.zeros((n, AUG - NHID - 1), jnp.float32)],
        axis=1)


def _layer1_body(x_ref, A_ref, W_ref, root_ref, b_ref, out_ref):
    h = _combine(x_ref[...], A_ref, W_ref, root_ref[...], b_ref[...])
    out_ref[...] = _augment(h)


def _layer1(x, A, W, root, b):
    return pl.pallas_call(
        _layer1_body,
        grid=(N_NODES // NODE_BLK,),
        in_specs=[
            pl.BlockSpec((NODE_BLK, N_FEAT), lambda i: (i, 0)),
            pl.BlockSpec((N_REL, NODE_BLK, AUG), lambda i: (0, i, 0)),
            pl.BlockSpec((N_REL, N_FEAT, NHID), lambda i: (0, 0, 0)),
            pl.BlockSpec((N_FEAT, NHID), lambda i: (0, 0)),
            pl.BlockSpec((1, NHID), lambda i: (0, 0)),
        ],
        out_specs=pl.BlockSpec((NODE_BLK, AUG), lambda i: (i, 0)),
        out_shape=jax.ShapeDtypeStruct((N_NODES, AUG), jnp.float32),
    )(x, A, W, root, b.reshape(1, NHID))


def _layer2_body(xa_ref, A_ref, W_ref, root_ref, b_ref, batch_ref, out_ref):
    h = _combine(xa_ref[:, :NHID], A_ref, W_ref, root_ref[...], b_ref[...])
    haug = _augment(h)
    bids = batch_ref[0, 0, :]
    oh = (bids[:, None] ==
          lax.broadcasted_iota(jnp.int32, (NODE_BLK, N_GRAPHS), 1)
          ).astype(jnp.float32)
    contrib = lax.dot_general(oh, haug, (((0,), (0,)), ((), ())),
                              preferred_element_type=jnp.float32)
    i = pl.program_id(0)

    @pl.when(i == 0)
    def _():
        out_ref[...] = contrib

    @pl.when(i > 0)
    def _():
        out_ref[...] = out_ref[...] + contrib


def _layer2_pool(h1a, A, W, root, b, batch):
    nblk = N_NODES // NODE_BLK
    return pl.pallas_call(
        _layer2_body,
        grid=(nblk,),
        in_specs=[
            pl.BlockSpec((NODE_BLK, AUG), lambda i: (i, 0)),
            pl.BlockSpec((N_REL, NODE_BLK, AUG), lambda i: (0, i, 0)),
            pl.BlockSpec((N_REL, NHID, NHID), lambda i: (0, 0, 0)),
            pl.BlockSpec((NHID, NHID), lambda i: (0, 0)),
            pl.BlockSpec((1, NHID), lambda i: (0, 0)),
            pl.BlockSpec((1, 1, NODE_BLK), lambda i: (i, 0, 0)),
        ],
        out_specs=pl.BlockSpec((N_GRAPHS, AUG), lambda i: (0, 0)),
        out_shape=jax.ShapeDtypeStruct((N_GRAPHS, AUG), jnp.float32),
    )(h1a, A, W, root, b.reshape(1, NHID), batch.reshape(nblk, 1, NODE_BLK))


def _head_body(p_ref, w1_ref, b1_ref, w2_ref, b2_ref, w3_ref, b3_ref, out_ref):
    p = p_ref[...]
    g = p[:, :NHID] / jnp.maximum(p[:, NHID:NHID + 1], 1.0)
    g = jnp.maximum(jnp.dot(g, w1_ref[...], preferred_element_type=jnp.float32)
                    + b1_ref[...], 0.0)
    g = jnp.maximum(jnp.dot(g, w2_ref[...], preferred_element_type=jnp.float32)
                    + b2_ref[...], 0.0)
    logits = jnp.dot(g, w3_ref[...], preferred_element_type=jnp.float32) + b3_ref[...]
    m = jnp.max(logits, axis=-1, keepdims=True)
    z = logits - m
    out_ref[...] = z - jnp.log(jnp.sum(jnp.exp(z), axis=-1, keepdims=True))


def _head(pooled, w1, b1, w2, b2, w3, b3):
    return pl.pallas_call(
        _head_body,
        out_shape=jax.ShapeDtypeStruct((N_GRAPHS, N_CLASSES), jnp.float32),
    )(pooled, w1, b1.reshape(1, NHID), w2, b2.reshape(1, NHID // 2),
      w3, b3.reshape(1, N_CLASSES))


def kernel(x, edge_index, edge_attr, batch, W1, root1, b1, W2, root2, b2,
           lin1_w, lin1_b, lin2_w, lin2_b, lin3_w, lin3_b):
    seg = _edge_seg(edge_attr, edge_index[1])
    pad = E_PAD - N_EDGES
    seg_p = jnp.concatenate([seg, jnp.full((pad,), 1 << 30, jnp.int32)])
    src_p = jnp.concatenate([edge_index[0], jnp.zeros((pad,), jnp.int32)])
    zs = jnp.zeros((SH_ROWS, AUG), jnp.float32)

    x_aug = _augment(x)
    A1 = _sc_scatter(x_aug, src_p, seg_p, zs)
    A1 = A1[:N_REL * N_NODES].reshape(N_REL, N_NODES, AUG)
    h1a = _layer1(x, A1, W1, root1, b1)

    A2 = _sc_scatter(h1a, src_p, seg_p, zs)
    A2 = A2[:N_REL * N_NODES].reshape(N_REL, N_NODES, AUG)
    pooled = _layer2_pool(h1a, A2, W2, root2, b2, batch)

    return _head(pooled, lin1_w, lin1_b, lin2_w, lin2_b, lin3_w, lin3_b)
